# baseline (speedup 1.0000x reference)
import functools

import jax
import jax.numpy as jnp
from jax import lax
from jax.experimental import pallas as pl
from jax.experimental.pallas import tpu as pltpu
from jax.experimental.pallas import tpu_sc as plsc

B = 4096
S = 200
D = 64
H = 256
C = 50
CPAD = 128

NC = 2
NS = 16
NW = NC * NS
BPW = B // NW
HALF = S // 2
LANES = 16
DCH = D // LANES


def _sc_bag(x3, table):
  mesh = plsc.VectorSubcoreMesh(core_axis_name="c", subcore_axis_name="s")

  @functools.partial(
      pl.kernel,
      out_type=jax.ShapeDtypeStruct((NW, BPW, D), jnp.float32),
      mesh=mesh,
      compiler_params=pltpu.CompilerParams(use_tc_tiling_on_sc=False),
      scratch_types=[
          pltpu.VMEM((BPW, 2, HALF), jnp.int32),
          pltpu.VMEM((HALF, D), jnp.float32),
          pltpu.VMEM((HALF, D), jnp.float32),
          pltpu.VMEM((BPW, D), jnp.float32),
          pltpu.SemaphoreType.DMA,
          pltpu.SemaphoreType.DMA,
      ],
  )
  def bag(x_hbm, tab_hbm, out_hbm, idx_v, buf_a, buf_b, out_v, sem_a, sem_b):
    w = lax.axis_index("s") * NC + lax.axis_index("c")
    pltpu.sync_copy(x_hbm.at[w], idx_v)

    def start(b, h, buf, sem):
      pltpu.async_copy(tab_hbm.at[idx_v.at[b, h]], buf, sem)

    def wait(buf, sem):
      pltpu.make_async_copy(tab_hbm.at[idx_v.at[0, 0]], buf, sem).wait()

    def accum(buf, accs):
      def body(r, accs):
        return tuple(
            accs[c] + buf[r, pl.ds(c * LANES, LANES)] for c in range(DCH)
        )
      return lax.fori_loop(0, HALF, body, accs)

    start(0, 0, buf_a, sem_a)
    start(0, 1, buf_b, sem_b)

    @pl.loop(0, BPW)
    def _(b):
      zeros = tuple(jnp.zeros((LANES,), jnp.float32) for _ in range(DCH))
      wait(buf_a, sem_a)
      acc = accum(buf_a, zeros)

      @pl.when(b < BPW - 1)
      def _():
        start(b + 1, 0, buf_a, sem_a)

      wait(buf_b, sem_b)
      acc = accum(buf_b, acc)

      @pl.when(b < BPW - 1)
      def _():
        start(b + 1, 1, buf_b, sem_b)

      for c in range(DCH):
        out_v[b, pl.ds(c * LANES, LANES)] = acc[c] * (1.0 / S)

    pltpu.sync_copy(out_v, out_hbm.at[w])

  return bag(x3, table)


def _mlp_body(x_ref, w1_ref, b1_ref, w2_ref, b2_ref, o_ref):
  x = x_ref[...]
  h = jnp.maximum(
      jnp.dot(x, w1_ref[...], preferred_element_type=jnp.float32) + b1_ref[...],
      0.0,
  )
  logits = (
      jnp.dot(h, w2_ref[...], preferred_element_type=jnp.float32) + b2_ref[...]
  )
  m = jnp.max(logits, axis=-1, keepdims=True)
  s = logits - m
  lse = jnp.log(jnp.sum(jnp.exp(s), axis=-1, keepdims=True))
  o_ref[...] = s - lse


def _mlp(bag, W1, b1, W2p, b2p):
  BB = 512
  return pl.pallas_call(
      _mlp_body,
      grid=(B // BB,),
      in_specs=[
          pl.BlockSpec((BB, D), lambda i: (i, 0)),
          pl.BlockSpec((D, H), lambda i: (0, 0)),
          pl.BlockSpec((1, H), lambda i: (0, 0)),
          pl.BlockSpec((H, CPAD), lambda i: (0, 0)),
          pl.BlockSpec((1, CPAD), lambda i: (0, 0)),
      ],
      out_specs=pl.BlockSpec((BB, CPAD), lambda i: (i, 0)),
      out_shape=jax.ShapeDtypeStruct((B, CPAD), jnp.float32),
  )(bag, W1, b1, W2p, b2p)


@jax.jit
def kernel(X, table, W1, b1, W2, b2):
  x3 = X.reshape(NW, BPW, 2, HALF)
  bag = _sc_bag(x3, table).reshape(B, D)
  W2p = jnp.pad(W2, ((0, 0), (0, CPAD - C)))
  b2p = jnp.pad(b2, (0, CPAD - C), constant_values=-1e30).reshape(1, CPAD)
  out = _mlp(bag, W1, b1.reshape(1, H), W2p, b2p)
  return out[:, :C]


# speedup vs baseline: 1.0003x; 1.0003x over previous
import functools

import jax
import jax.numpy as jnp
from jax import lax
from jax.experimental import pallas as pl
from jax.experimental.pallas import tpu as pltpu
from jax.experimental.pallas import tpu_sc as plsc

B = 4096
S = 200
D = 64
H = 256
C = 50
CPAD = 128

NC = 2
NS = 16
NW = NC * NS
BPW = B // NW
SA = 104
SB = S - SA
LANES = 16
DCH = D // LANES


def _sc_bag(X, table):
  mesh = plsc.VectorSubcoreMesh(core_axis_name="c", subcore_axis_name="s")

  @functools.partial(
      pl.kernel,
      out_type=jax.ShapeDtypeStruct((B, D), jnp.float32),
      mesh=mesh,
      compiler_params=pltpu.CompilerParams(use_tc_tiling_on_sc=False),
      scratch_types=[
          pltpu.VMEM((BPW, S), jnp.int32),
          pltpu.VMEM((SA, D), jnp.float32),
          pltpu.VMEM((SB, D), jnp.float32),
          pltpu.VMEM((BPW, D), jnp.float32),
          pltpu.SemaphoreType.DMA,
          pltpu.SemaphoreType.DMA,
      ],
  )
  def bag(x_hbm, tab_hbm, out_hbm, idx_v, buf_a, buf_b, out_v, sem_a, sem_b):
    w = lax.axis_index("s") * NC + lax.axis_index("c")
    base = w * BPW
    pltpu.sync_copy(x_hbm.at[pl.ds(base, BPW)], idx_v)

    def start_a(b):
      pltpu.async_copy(tab_hbm.at[idx_v.at[b, pl.ds(0, SA)]], buf_a, sem_a)

    def start_b(b):
      pltpu.async_copy(tab_hbm.at[idx_v.at[b, pl.ds(SA, SB)]], buf_b, sem_b)

    def wait(idx_slice, buf, sem):
      pltpu.make_async_copy(tab_hbm.at[idx_slice], buf, sem).wait()

    def accum(buf, n, accs):
      def body(r, accs):
        return tuple(
            accs[c] + buf[r, pl.ds(c * LANES, LANES)] for c in range(DCH)
        )
      return lax.fori_loop(0, n, body, accs)

    start_a(0)
    start_b(0)

    @pl.loop(0, BPW)
    def _(b):
      zeros = tuple(jnp.zeros((LANES,), jnp.float32) for _ in range(DCH))
      wait(idx_v.at[0, pl.ds(0, SA)], buf_a, sem_a)
      acc = accum(buf_a, SA, zeros)

      @pl.when(b < BPW - 1)
      def _():
        start_a(b + 1)

      wait(idx_v.at[0, pl.ds(SA, SB)], buf_b, sem_b)
      acc = accum(buf_b, SB, acc)

      @pl.when(b < BPW - 1)
      def _():
        start_b(b + 1)

      for c in range(DCH):
        out_v[b, pl.ds(c * LANES, LANES)] = acc[c] * (1.0 / S)

    pltpu.sync_copy(out_v, out_hbm.at[pl.ds(base, BPW)])

  return bag(X, table)


def _mlp_body(x_ref, w1_ref, b1_ref, w2_ref, b2_ref, o_ref):
  x = x_ref[...]
  h = jnp.maximum(
      jnp.dot(x, w1_ref[...], preferred_element_type=jnp.float32) + b1_ref[...],
      0.0,
  )
  logits = (
      jnp.dot(h, w2_ref[...], preferred_element_type=jnp.float32) + b2_ref[...]
  )
  m = jnp.max(logits, axis=-1, keepdims=True)
  s = logits - m
  lse = jnp.log(jnp.sum(jnp.exp(s), axis=-1, keepdims=True))
  o_ref[...] = s - lse


def _mlp(bag, W1, b1, W2p, b2p):
  BB = 512
  return pl.pallas_call(
      _mlp_body,
      grid=(B // BB,),
      in_specs=[
          pl.BlockSpec((BB, D), lambda i: (i, 0)),
          pl.BlockSpec((D, H), lambda i: (0, 0)),
          pl.BlockSpec((1, H), lambda i: (0, 0)),
          pl.BlockSpec((H, CPAD), lambda i: (0, 0)),
          pl.BlockSpec((1, CPAD), lambda i: (0, 0)),
      ],
      out_specs=pl.BlockSpec((BB, CPAD), lambda i: (i, 0)),
      out_shape=jax.ShapeDtypeStruct((B, CPAD), jnp.float32),
  )(bag, W1, b1, W2p, b2p)


@jax.jit
def kernel(X, table, W1, b1, W2, b2):
  bag = _sc_bag(X, table)
  W2p = jnp.pad(W2, ((0, 0), (0, CPAD - C)))
  b2p = jnp.pad(b2, (0, CPAD - C), constant_values=-1e30).reshape(1, CPAD)
  out = _mlp(bag, W1, b1.reshape(1, H), W2p, b2p)
  return out[:, :C]


# speedup vs baseline: 1.0009x; 1.0006x over previous
import functools

import jax
import jax.numpy as jnp
from jax import lax
from jax.experimental import pallas as pl
from jax.experimental.pallas import tpu as pltpu
from jax.experimental.pallas import tpu_sc as plsc

B = 4096
S = 200
D = 64
H = 256
C = 50
CPAD = 128

NC = 2
NS = 16
NW = NC * NS
BPW = B // NW
SA = 104
SB = S - SA
SP = 256
LANES = 16
DCH = D // LANES


def _sc_bag(X, table):
  mesh = plsc.VectorSubcoreMesh(core_axis_name="c", subcore_axis_name="s")

  @functools.partial(
      pl.kernel,
      out_type=jax.ShapeDtypeStruct((B, D), jnp.float32),
      mesh=mesh,
      compiler_params=pltpu.CompilerParams(use_tc_tiling_on_sc=False),
      scratch_types=[
          pltpu.VMEM((BPW, SP), jnp.int32),
          pltpu.VMEM((SA, D), jnp.float32),
          pltpu.VMEM((SB, D), jnp.float32),
          pltpu.VMEM((BPW, D), jnp.float32),
          pltpu.SemaphoreType.DMA,
          pltpu.SemaphoreType.DMA,
      ],
  )
  def bag(x_hbm, tab_hbm, out_hbm, idx_v, buf_a, buf_b, out_v, sem_a, sem_b):
    w = lax.axis_index("s") * NC + lax.axis_index("c")
    base = w * BPW
    pltpu.sync_copy(x_hbm.at[pl.ds(base, BPW)], idx_v)

    def start_a(b):
      pltpu.async_copy(tab_hbm.at[idx_v.at[b, pl.ds(0, SA)]], buf_a, sem_a)

    def start_b(b):
      pltpu.async_copy(tab_hbm.at[idx_v.at[b, pl.ds(SA, SB)]], buf_b, sem_b)

    def wait(idx_slice, buf, sem):
      pltpu.make_async_copy(tab_hbm.at[idx_slice], buf, sem).wait()

    def accum(buf, n, accs):
      def body(r, accs):
        return tuple(
            accs[c] + buf[r, pl.ds(c * LANES, LANES)] for c in range(DCH)
        )
      return lax.fori_loop(0, n, body, accs)

    start_a(0)
    start_b(0)

    @pl.loop(0, BPW)
    def _(b):
      zeros = tuple(jnp.zeros((LANES,), jnp.float32) for _ in range(DCH))
      wait(idx_v.at[0, pl.ds(0, SA)], buf_a, sem_a)
      acc = accum(buf_a, SA, zeros)

      @pl.when(b < BPW - 1)
      def _():
        start_a(b + 1)

      wait(idx_v.at[0, pl.ds(SA, SB)], buf_b, sem_b)
      acc = accum(buf_b, SB, acc)

      @pl.when(b < BPW - 1)
      def _():
        start_b(b + 1)

      for c in range(DCH):
        out_v[b, pl.ds(c * LANES, LANES)] = acc[c] * (1.0 / S)

    pltpu.sync_copy(out_v, out_hbm.at[pl.ds(base, BPW)])

  return bag(X, table)


def _mlp_body(x_ref, w1_ref, b1_ref, w2_ref, b2_ref, o_ref):
  x = x_ref[...]
  h = jnp.maximum(
      jnp.dot(x, w1_ref[...], preferred_element_type=jnp.float32) + b1_ref[...],
      0.0,
  )
  logits = (
      jnp.dot(h, w2_ref[...], preferred_element_type=jnp.float32) + b2_ref[...]
  )
  m = jnp.max(logits, axis=-1, keepdims=True)
  s = logits - m
  lse = jnp.log(jnp.sum(jnp.exp(s), axis=-1, keepdims=True))
  o_ref[...] = s - lse


def _mlp(bag, W1, b1, W2p, b2p):
  BB = 512
  return pl.pallas_call(
      _mlp_body,
      grid=(B // BB,),
      in_specs=[
          pl.BlockSpec((BB, D), lambda i: (i, 0)),
          pl.BlockSpec((D, H), lambda i: (0, 0)),
          pl.BlockSpec((1, H), lambda i: (0, 0)),
          pl.BlockSpec((H, CPAD), lambda i: (0, 0)),
          pl.BlockSpec((1, CPAD), lambda i: (0, 0)),
      ],
      out_specs=pl.BlockSpec((BB, CPAD), lambda i: (i, 0)),
      out_shape=jax.ShapeDtypeStruct((B, CPAD), jnp.float32),
  )(bag, W1, b1, W2p, b2p)


@jax.jit
def kernel(X, table, W1, b1, W2, b2):
  Xp = jnp.pad(X, ((0, 0), (0, SP - S)))
  bag = _sc_bag(Xp, table)
  W2p = jnp.pad(W2, ((0, 0), (0, CPAD - C)))
  b2p = jnp.pad(b2, (0, CPAD - C), constant_values=-1e30).reshape(1, CPAD)
  out = _mlp(bag, W1, b1.reshape(1, H), W2p, b2p)
  return out[:, :C]
